# baseline (speedup 1.0000x reference)
import functools

import jax
import jax.numpy as jnp
from jax import lax
from jax.experimental import pallas as pl
from jax.experimental.pallas import tpu as pltpu
from jax.experimental.pallas import tpu_sc as plsc

_N = 10000
_E = 160000
_ND = 256
_ED = 16
_HD = 256
_OD = 128
_EPS = 1e-5

_BN = 2000
_BE = 2000


def _nt_kernel(x_ref, w_ref, b_ref, o_ref):
    o_ref[...] = (
        jnp.dot(x_ref[...], w_ref[...], preferred_element_type=jnp.float32)
        + b_ref[...]
    )


def _node_transform(x, w, b):
    n = x.shape[0]
    f = w.shape[1]
    return pl.pallas_call(
        _nt_kernel,
        grid=(n // _BN,),
        in_specs=[
            pl.BlockSpec((_BN, _ND), lambda i: (i, 0)),
            pl.BlockSpec((_ND, f), lambda i: (0, 0)),
            pl.BlockSpec((1, f), lambda i: (0, 0)),
        ],
        out_specs=pl.BlockSpec((_BN, f), lambda i: (i, 0)),
        out_shape=jax.ShapeDtypeStruct((n, f), jnp.float32),
    )(x, w, b)


_SC_NC = 2
_SC_NS = 16
_SC_NW = _SC_NC * _SC_NS
_SC_CH = 40


def _make_sc_gather(d):
    b_per_w = _E // _SC_NW
    n_chunks = b_per_w // _SC_CH
    n_pairs = n_chunks // 2
    mesh = plsc.VectorSubcoreMesh(core_axis_name="c", subcore_axis_name="s")

    @functools.partial(
        pl.kernel,
        mesh=mesh,
        out_type=jax.ShapeDtypeStruct((_E, d), jnp.float32),
        scratch_types=[
            pltpu.VMEM((_SC_CH,), jnp.int32),
            pltpu.VMEM((_SC_CH,), jnp.int32),
            pltpu.VMEM((_SC_CH, d), jnp.float32),
            pltpu.VMEM((_SC_CH, d), jnp.float32),
            pltpu.SemaphoreType.DMA,
            pltpu.SemaphoreType.DMA,
        ],
    )
    def gather_kernel(
        table_hbm, idx_hbm, out_hbm, idx0, idx1, rows0, rows1, sem0, sem1
    ):
        wid = lax.axis_index("s") * _SC_NC + lax.axis_index("c")
        base = wid * b_per_w

        def body(p, _):
            off0 = base + (2 * p) * _SC_CH
            off1 = off0 + _SC_CH
            pltpu.sync_copy(idx_hbm.at[pl.ds(off0, _SC_CH)], idx0)
            cp0 = pltpu.async_copy(table_hbm.at[idx0], rows0, sem0)
            pltpu.sync_copy(idx_hbm.at[pl.ds(off1, _SC_CH)], idx1)
            cp1 = pltpu.async_copy(table_hbm.at[idx1], rows1, sem1)
            cp0.wait()
            pltpu.sync_copy(rows0, out_hbm.at[pl.ds(off0, _SC_CH)])
            cp1.wait()
            pltpu.sync_copy(rows1, out_hbm.at[pl.ds(off1, _SC_CH)])
            return _

        lax.fori_loop(0, n_pairs, body, None)
        if n_chunks % 2:
            off = base + (n_chunks - 1) * _SC_CH
            pltpu.sync_copy(idx_hbm.at[pl.ds(off, _SC_CH)], idx0)
            pltpu.async_copy(table_hbm.at[idx0], rows0, sem0).wait()
            pltpu.sync_copy(rows0, out_hbm.at[pl.ds(off, _SC_CH)])

    return gather_kernel


def _make_sc_gather2(d0, d1):
    b_per_w = _E // _SC_NW
    n_chunks = b_per_w // _SC_CH
    mesh = plsc.VectorSubcoreMesh(core_axis_name="c", subcore_axis_name="s")

    @functools.partial(
        pl.kernel,
        mesh=mesh,
        out_type=[
            jax.ShapeDtypeStruct((_E, d0), jnp.float32),
            jax.ShapeDtypeStruct((_E, d1), jnp.float32),
        ],
        scratch_types=[
            pltpu.VMEM((_SC_CH,), jnp.int32),
            pltpu.VMEM((_SC_CH,), jnp.int32),
            pltpu.VMEM((_SC_CH, d0), jnp.float32),
            pltpu.VMEM((_SC_CH, d1), jnp.float32),
            pltpu.SemaphoreType.DMA,
            pltpu.SemaphoreType.DMA,
        ],
    )
    def gather_kernel(
        t0_hbm, t1_hbm, i0_hbm, i1_hbm, o0_hbm, o1_hbm,
        idx0, idx1, rows0, rows1, sem0, sem1,
    ):
        wid = lax.axis_index("s") * _SC_NC + lax.axis_index("c")
        base = wid * b_per_w

        def body(c, _):
            off = base + c * _SC_CH
            pltpu.sync_copy(i0_hbm.at[pl.ds(off, _SC_CH)], idx0)
            cp0 = pltpu.async_copy(t0_hbm.at[idx0], rows0, sem0)
            pltpu.sync_copy(i1_hbm.at[pl.ds(off, _SC_CH)], idx1)
            cp1 = pltpu.async_copy(t1_hbm.at[idx1], rows1, sem1)
            cp0.wait()
            pltpu.sync_copy(rows0, o0_hbm.at[pl.ds(off, _SC_CH)])
            cp1.wait()
            pltpu.sync_copy(rows1, o1_hbm.at[pl.ds(off, _SC_CH)])
            return _

        lax.fori_loop(0, n_chunks, body, None)

    return gather_kernel


_sc_gather_pair = _make_sc_gather2(3 * _HD, 2 * _HD)


def _edge_a_kernel(rowg, colg, ea, w1e, b1, h1, s, hsum, hsq):
    i = pl.program_id(0)
    rg = rowg[...]
    cg = colg[...]
    h = (
        rg[:, 0:_HD]
        + cg[:, 0:_HD]
        + jnp.dot(ea[...], w1e[...], preferred_element_type=jnp.float32)
        + b1[...]
    )
    h1[...] = h
    s[...] = jnp.sum(
        rg[:, _HD : 2 * _HD] * cg[:, _HD : 2 * _HD], axis=1, keepdims=True
    )

    @pl.when(i == 0)
    def _init():
        hsum[...] = jnp.zeros_like(hsum)
        hsq[...] = jnp.zeros_like(hsq)

    hsum[...] += jnp.sum(h, axis=0, keepdims=True)
    hsq[...] += jnp.sum(h * h, axis=0, keepdims=True)


def _edge_a(rowg, colg, ea, w1e, b1):
    return pl.pallas_call(
        _edge_a_kernel,
        grid=(_E // _BE,),
        in_specs=[
            pl.BlockSpec((_BE, 3 * _HD), lambda i: (i, 0)),
            pl.BlockSpec((_BE, 2 * _HD), lambda i: (i, 0)),
            pl.BlockSpec((_BE, _ED), lambda i: (i, 0)),
            pl.BlockSpec((_ED, _HD), lambda i: (0, 0)),
            pl.BlockSpec((1, _HD), lambda i: (0, 0)),
        ],
        out_specs=[
            pl.BlockSpec((_BE, _HD), lambda i: (i, 0)),
            pl.BlockSpec((_BE, 1), lambda i: (i, 0)),
            pl.BlockSpec((1, _HD), lambda i: (0, 0)),
            pl.BlockSpec((1, _HD), lambda i: (0, 0)),
        ],
        out_shape=[
            jax.ShapeDtypeStruct((_E, _HD), jnp.float32),
            jax.ShapeDtypeStruct((_E, 1), jnp.float32),
            jax.ShapeDtypeStruct((1, _HD), jnp.float32),
            jax.ShapeDtypeStruct((1, _HD), jnp.float32),
        ],
    )(rowg, colg, ea, w1e, b1)


def _edge_b_kernel(
    h1, hsum, hsq, w2e, b2e, wn1e, bn1, rowg, s, smax_g,
    eout, g, ex, gsum, gsq,
):
    i = pl.program_id(0)
    m = hsum[...] / _E
    v = hsq[...] / _E - m * m
    inv = jax.lax.rsqrt(v + _EPS)
    e1 = jnp.maximum((h1[...] - m) * inv, 0.0)
    eo = jnp.dot(e1, w2e[...], preferred_element_type=jnp.float32) + b2e[...]
    eout[...] = eo
    gb = (
        rowg[...][:, 2 * _HD :]
        + jnp.dot(eo, wn1e[...], preferred_element_type=jnp.float32)
        + bn1[...]
    )
    g[...] = gb
    ex[...] = jnp.exp(s[...] - smax_g[...])

    @pl.when(i == 0)
    def _init():
        gsum[...] = jnp.zeros_like(gsum)
        gsq[...] = jnp.zeros_like(gsq)

    gsum[...] += jnp.sum(gb, axis=0, keepdims=True)
    gsq[...] += jnp.sum(gb * gb, axis=0, keepdims=True)


def _edge_b(h1, hsum, hsq, w2e, b2e, wn1e, bn1, rowg, s, smax_g):
    return pl.pallas_call(
        _edge_b_kernel,
        grid=(_E // _BE,),
        in_specs=[
            pl.BlockSpec((_BE, _HD), lambda i: (i, 0)),
            pl.BlockSpec((1, _HD), lambda i: (0, 0)),
            pl.BlockSpec((1, _HD), lambda i: (0, 0)),
            pl.BlockSpec((_HD, _ED), lambda i: (0, 0)),
            pl.BlockSpec((1, _ED), lambda i: (0, 0)),
            pl.BlockSpec((_ED, _HD), lambda i: (0, 0)),
            pl.BlockSpec((1, _HD), lambda i: (0, 0)),
            pl.BlockSpec((_BE, 3 * _HD), lambda i: (i, 0)),
            pl.BlockSpec((_BE, 1), lambda i: (i, 0)),
            pl.BlockSpec((_BE, 1), lambda i: (i, 0)),
        ],
        out_specs=[
            pl.BlockSpec((_BE, _ED), lambda i: (i, 0)),
            pl.BlockSpec((_BE, _HD), lambda i: (i, 0)),
            pl.BlockSpec((_BE, 1), lambda i: (i, 0)),
            pl.BlockSpec((1, _HD), lambda i: (0, 0)),
            pl.BlockSpec((1, _HD), lambda i: (0, 0)),
        ],
        out_shape=[
            jax.ShapeDtypeStruct((_E, _ED), jnp.float32),
            jax.ShapeDtypeStruct((_E, _HD), jnp.float32),
            jax.ShapeDtypeStruct((_E, 1), jnp.float32),
            jax.ShapeDtypeStruct((1, _HD), jnp.float32),
            jax.ShapeDtypeStruct((1, _HD), jnp.float32),
        ],
    )(h1, hsum, hsq, w2e, b2e, wn1e, bn1, rowg, s, smax_g)


def _edge_c_kernel(g, gsum, gsq, wn2, bn2, ex, den_g, hw, attn):
    m = gsum[...] / _E
    v = gsq[...] / _E - m * m
    inv = jax.lax.rsqrt(v + _EPS)
    gn = jnp.maximum((g[...] - m) * inv, 0.0)
    h = jnp.dot(gn, wn2[...], preferred_element_type=jnp.float32) + bn2[...]
    a = ex[...] / den_g[...]
    attn[...] = a
    hw[...] = h * a


def _edge_c(g, gsum, gsq, wn2, bn2, ex, den_g):
    return pl.pallas_call(
        _edge_c_kernel,
        grid=(_E // _BE,),
        in_specs=[
            pl.BlockSpec((_BE, _HD), lambda i: (i, 0)),
            pl.BlockSpec((1, _HD), lambda i: (0, 0)),
            pl.BlockSpec((1, _HD), lambda i: (0, 0)),
            pl.BlockSpec((_HD, _HD), lambda i: (0, 0)),
            pl.BlockSpec((1, _HD), lambda i: (0, 0)),
            pl.BlockSpec((_BE, 1), lambda i: (i, 0)),
            pl.BlockSpec((_BE, 1), lambda i: (i, 0)),
        ],
        out_specs=[
            pl.BlockSpec((_BE, _HD), lambda i: (i, 0)),
            pl.BlockSpec((_BE, 1), lambda i: (i, 0)),
        ],
        out_shape=[
            jax.ShapeDtypeStruct((_E, _HD), jnp.float32),
            jax.ShapeDtypeStruct((_E, 1), jnp.float32),
        ],
    )(g, gsum, gsq, wn2, bn2, ex, den_g)


def _node_a_kernel(x, agg, wm1x, wm1a, bm1, t, tsum, tsq):
    i = pl.program_id(0)
    tb = (
        jnp.dot(x[...], wm1x[...], preferred_element_type=jnp.float32)
        + jnp.dot(agg[...], wm1a[...], preferred_element_type=jnp.float32)
        + bm1[...]
    )
    t[...] = tb

    @pl.when(i == 0)
    def _init():
        tsum[...] = jnp.zeros_like(tsum)
        tsq[...] = jnp.zeros_like(tsq)

    tsum[...] += jnp.sum(tb, axis=0, keepdims=True)
    tsq[...] += jnp.sum(tb * tb, axis=0, keepdims=True)


def _node_a(x, agg, wm1x, wm1a, bm1):
    return pl.pallas_call(
        _node_a_kernel,
        grid=(_N // _BN,),
        in_specs=[
            pl.BlockSpec((_BN, _ND), lambda i: (i, 0)),
            pl.BlockSpec((_BN, _HD), lambda i: (i, 0)),
            pl.BlockSpec((_ND, _HD), lambda i: (0, 0)),
            pl.BlockSpec((_HD, _HD), lambda i: (0, 0)),
            pl.BlockSpec((1, _HD), lambda i: (0, 0)),
        ],
        out_specs=[
            pl.BlockSpec((_BN, _HD), lambda i: (i, 0)),
            pl.BlockSpec((1, _HD), lambda i: (0, 0)),
            pl.BlockSpec((1, _HD), lambda i: (0, 0)),
        ],
        out_shape=[
            jax.ShapeDtypeStruct((_N, _HD), jnp.float32),
            jax.ShapeDtypeStruct((1, _HD), jnp.float32),
            jax.ShapeDtypeStruct((1, _HD), jnp.float32),
        ],
    )(x, agg, wm1x, wm1a, bm1)


def _node_b_kernel(t, tsum, tsq, wm2, bm2, xout):
    m = tsum[...] / _N
    v = tsq[...] / _N - m * m
    inv = jax.lax.rsqrt(v + _EPS)
    tn = jnp.maximum((t[...] - m) * inv, 0.0)
    xout[...] = (
        jnp.dot(tn, wm2[...], preferred_element_type=jnp.float32) + bm2[...]
    )


def _node_b(t, tsum, tsq, wm2, bm2):
    return pl.pallas_call(
        _node_b_kernel,
        grid=(_N // _BN,),
        in_specs=[
            pl.BlockSpec((_BN, _HD), lambda i: (i, 0)),
            pl.BlockSpec((1, _HD), lambda i: (0, 0)),
            pl.BlockSpec((1, _HD), lambda i: (0, 0)),
            pl.BlockSpec((_HD, _ND), lambda i: (0, 0)),
            pl.BlockSpec((1, _ND), lambda i: (0, 0)),
        ],
        out_specs=pl.BlockSpec((_BN, _ND), lambda i: (i, 0)),
        out_shape=jax.ShapeDtypeStruct((_N, _ND), jnp.float32),
    )(t, tsum, tsq, wm2, bm2)


def _proj_kernel(x, wp, bp, emb, xsum, gemb):
    i = pl.program_id(0)
    emb[...] = jnp.maximum(
        jnp.dot(x[...], wp[...], preferred_element_type=jnp.float32) + bp[...],
        0.0,
    )

    @pl.when(i == 0)
    def _init():
        xsum[...] = jnp.zeros_like(xsum)
        gemb[...] = jnp.zeros_like(gemb)

    xsum[...] += jnp.sum(x[...], axis=0, keepdims=True)

    @pl.when(i == pl.num_programs(0) - 1)
    def _final():
        gemb[...] = jnp.maximum(
            jnp.dot(
                xsum[...] / _N, wp[...], preferred_element_type=jnp.float32
            )
            + bp[...],
            0.0,
        )


def _proj(x, wp, bp):
    return pl.pallas_call(
        _proj_kernel,
        grid=(_N // _BN,),
        in_specs=[
            pl.BlockSpec((_BN, _ND), lambda i: (i, 0)),
            pl.BlockSpec((_ND, _OD), lambda i: (0, 0)),
            pl.BlockSpec((1, _OD), lambda i: (0, 0)),
        ],
        out_specs=[
            pl.BlockSpec((_BN, _OD), lambda i: (i, 0)),
            pl.BlockSpec((1, _ND), lambda i: (0, 0)),
            pl.BlockSpec((1, _OD), lambda i: (0, 0)),
        ],
        out_shape=[
            jax.ShapeDtypeStruct((_N, _OD), jnp.float32),
            jax.ShapeDtypeStruct((1, _ND), jnp.float32),
            jax.ShapeDtypeStruct((1, _OD), jnp.float32),
        ],
    )(x, wp, bp)


def _layer(x, edge_attr, lp, row, col):
    (w_e1, b_e1), (w_e2, b_e2) = lp["edge_mlp"]
    wq, bq = lp["attn_q"]
    wk, bk = lp["attn_k"]
    (w_n1, b_n1), (w_n2, b_n2) = lp["node_mlp1"]
    (w_m1, b_m1), (w_m2, b_m2) = lp["node_mlp2"]

    w1s = w_e1[:_ND]
    w1d = w_e1[_ND : 2 * _ND]
    w1e = w_e1[2 * _ND :]
    wn1x = w_n1[:_ND]
    wn1e = w_n1[_ND:]
    wm1x = w_m1[:_ND]
    wm1a = w_m1[_ND:]

    z = jnp.zeros((_HD,), jnp.float32)
    w_row = jnp.concatenate([w1s, wq, wn1x], axis=1)
    b_row = jnp.concatenate([z, bq, z])[None]
    w_col = jnp.concatenate([w1d, wk], axis=1)
    b_col = jnp.concatenate([z, bk])[None]
    xt_row = _node_transform(x, w_row, b_row)
    xt_col = _node_transform(x, w_col, b_col)

    rowg, colg = _sc_gather_pair(xt_row, xt_col, row, col)

    h1, s, hsum, hsq = _edge_a(rowg, colg, edge_attr, w1e, b_e1[None])

    s1 = s[:, 0]
    smax = jax.ops.segment_max(s1, row, num_segments=_N)
    smax_g = jnp.take(smax, row, axis=0)[:, None]

    eout, g, ex, gsum, gsq = _edge_b(
        h1, hsum, hsq, w_e2, b_e2[None], wn1e, b_n1[None], rowg, s, smax_g
    )

    den = jax.ops.segment_sum(ex[:, 0], row, num_segments=_N)
    den_g = jnp.take(den, row, axis=0)[:, None]

    hw, attn = _edge_c(g, gsum, gsq, w_n2, b_n2[None], ex, den_g)

    agg = jax.ops.segment_sum(hw, col, num_segments=_N)

    t, tsum, tsq = _node_a(x, agg, wm1x, wm1a, b_m1[None])
    x_out = _node_b(t, tsum, tsq, wm2=w_m2, bm2=b_m2[None])

    return x_out, eout, attn[:, 0]


def kernel(x, edge_index, edge_attr, params):
    row = edge_index[0].astype(jnp.int32)
    col = edge_index[1].astype(jnp.int32)
    attentions = []
    for lp in params["layers"]:
        x, edge_attr, attn = _layer(x, edge_attr, lp, row, col)
        attentions.append(attn)
    wp, bp = params["proj"]
    emb, _, gemb = _proj(x, wp, bp[None])
    u = params["layers"][-1]["global_mlp"][1][1][None, :]
    return (gemb, (emb, x, edge_attr, u, tuple(attentions)))


# speedup vs baseline: 1.0213x; 1.0213x over previous
import functools

import jax
import jax.numpy as jnp
from jax import lax
from jax.experimental import pallas as pl
from jax.experimental.pallas import tpu as pltpu
from jax.experimental.pallas import tpu_sc as plsc

_N = 10000
_E = 160000
_ND = 256
_ED = 16
_HD = 256
_OD = 128
_EPS = 1e-5

_BN = 2000
_BE = 2000


def _nt_kernel(x_ref, w_ref, b_ref, o_ref):
    o_ref[...] = (
        jnp.dot(x_ref[...], w_ref[...], preferred_element_type=jnp.float32)
        + b_ref[...]
    )


def _node_transform(x, w, b):
    n = x.shape[0]
    f = w.shape[1]
    return pl.pallas_call(
        _nt_kernel,
        grid=(n // _BN,),
        in_specs=[
            pl.BlockSpec((_BN, _ND), lambda i: (i, 0)),
            pl.BlockSpec((_ND, f), lambda i: (0, 0)),
            pl.BlockSpec((1, f), lambda i: (0, 0)),
        ],
        out_specs=pl.BlockSpec((_BN, f), lambda i: (i, 0)),
        out_shape=jax.ShapeDtypeStruct((n, f), jnp.float32),
    )(x, w, b)


_SC_NC = 2
_SC_NS = 16
_SC_NW = _SC_NC * _SC_NS
_SC_CH = 40


def _make_sc_gather(d):
    b_per_w = _E // _SC_NW
    n_chunks = b_per_w // _SC_CH
    n_pairs = n_chunks // 2
    mesh = plsc.VectorSubcoreMesh(core_axis_name="c", subcore_axis_name="s")

    @functools.partial(
        pl.kernel,
        mesh=mesh,
        out_type=jax.ShapeDtypeStruct((_E, d), jnp.float32),
        scratch_types=[
            pltpu.VMEM((_SC_CH,), jnp.int32),
            pltpu.VMEM((_SC_CH,), jnp.int32),
            pltpu.VMEM((_SC_CH, d), jnp.float32),
            pltpu.VMEM((_SC_CH, d), jnp.float32),
            pltpu.SemaphoreType.DMA,
            pltpu.SemaphoreType.DMA,
        ],
    )
    def gather_kernel(
        table_hbm, idx_hbm, out_hbm, idx0, idx1, rows0, rows1, sem0, sem1
    ):
        wid = lax.axis_index("s") * _SC_NC + lax.axis_index("c")
        base = wid * b_per_w

        def body(p, _):
            off0 = base + (2 * p) * _SC_CH
            off1 = off0 + _SC_CH
            pltpu.sync_copy(idx_hbm.at[pl.ds(off0, _SC_CH)], idx0)
            cp0 = pltpu.async_copy(table_hbm.at[idx0], rows0, sem0)
            pltpu.sync_copy(idx_hbm.at[pl.ds(off1, _SC_CH)], idx1)
            cp1 = pltpu.async_copy(table_hbm.at[idx1], rows1, sem1)
            cp0.wait()
            pltpu.sync_copy(rows0, out_hbm.at[pl.ds(off0, _SC_CH)])
            cp1.wait()
            pltpu.sync_copy(rows1, out_hbm.at[pl.ds(off1, _SC_CH)])
            return _

        lax.fori_loop(0, n_pairs, body, None)
        if n_chunks % 2:
            off = base + (n_chunks - 1) * _SC_CH
            pltpu.sync_copy(idx_hbm.at[pl.ds(off, _SC_CH)], idx0)
            pltpu.async_copy(table_hbm.at[idx0], rows0, sem0).wait()
            pltpu.sync_copy(rows0, out_hbm.at[pl.ds(off, _SC_CH)])

    return gather_kernel


def _make_sc_gather2(d0, d1):
    b_per_w = _E // _SC_NW
    n_chunks = b_per_w // _SC_CH
    mesh = plsc.VectorSubcoreMesh(core_axis_name="c", subcore_axis_name="s")

    @functools.partial(
        pl.kernel,
        mesh=mesh,
        out_type=[
            jax.ShapeDtypeStruct((_E, d0), jnp.float32),
            jax.ShapeDtypeStruct((_E, d1), jnp.float32),
        ],
        scratch_types=[
            pltpu.VMEM((_SC_CH,), jnp.int32),
            pltpu.VMEM((_SC_CH,), jnp.int32),
            pltpu.VMEM((_SC_CH,), jnp.int32),
            pltpu.VMEM((_SC_CH,), jnp.int32),
            pltpu.VMEM((_SC_CH, d0), jnp.float32),
            pltpu.VMEM((_SC_CH, d0), jnp.float32),
            pltpu.VMEM((_SC_CH, d1), jnp.float32),
            pltpu.VMEM((_SC_CH, d1), jnp.float32),
            pltpu.SemaphoreType.DMA,
            pltpu.SemaphoreType.DMA,
            pltpu.SemaphoreType.DMA,
            pltpu.SemaphoreType.DMA,
        ],
    )
    def gather_kernel(
        t0_hbm, t1_hbm, i0_hbm, i1_hbm, o0_hbm, o1_hbm,
        i0a, i0b, i1a, i1b, r0a, r0b, r1a, r1b, s0a, s0b, s1a, s1b,
    ):
        wid = lax.axis_index("s") * _SC_NC + lax.axis_index("c")
        base = wid * b_per_w
        n_pairs = n_chunks // 2

        def body(p, _):
            offa = base + (2 * p) * _SC_CH
            offb = offa + _SC_CH
            pltpu.sync_copy(i0_hbm.at[pl.ds(offa, _SC_CH)], i0a)
            cp0a = pltpu.async_copy(t0_hbm.at[i0a], r0a, s0a)
            pltpu.sync_copy(i1_hbm.at[pl.ds(offa, _SC_CH)], i1a)
            cp1a = pltpu.async_copy(t1_hbm.at[i1a], r1a, s1a)
            pltpu.sync_copy(i0_hbm.at[pl.ds(offb, _SC_CH)], i0b)
            cp0b = pltpu.async_copy(t0_hbm.at[i0b], r0b, s0b)
            pltpu.sync_copy(i1_hbm.at[pl.ds(offb, _SC_CH)], i1b)
            cp1b = pltpu.async_copy(t1_hbm.at[i1b], r1b, s1b)
            cp0a.wait()
            pltpu.sync_copy(r0a, o0_hbm.at[pl.ds(offa, _SC_CH)])
            cp1a.wait()
            pltpu.sync_copy(r1a, o1_hbm.at[pl.ds(offa, _SC_CH)])
            cp0b.wait()
            pltpu.sync_copy(r0b, o0_hbm.at[pl.ds(offb, _SC_CH)])
            cp1b.wait()
            pltpu.sync_copy(r1b, o1_hbm.at[pl.ds(offb, _SC_CH)])
            return _

        lax.fori_loop(0, n_pairs, body, None)
        if n_chunks % 2:
            off = base + (n_chunks - 1) * _SC_CH
            pltpu.sync_copy(i0_hbm.at[pl.ds(off, _SC_CH)], i0a)
            cp0 = pltpu.async_copy(t0_hbm.at[i0a], r0a, s0a)
            pltpu.sync_copy(i1_hbm.at[pl.ds(off, _SC_CH)], i1a)
            cp1 = pltpu.async_copy(t1_hbm.at[i1a], r1a, s1a)
            cp0.wait()
            pltpu.sync_copy(r0a, o0_hbm.at[pl.ds(off, _SC_CH)])
            cp1.wait()
            pltpu.sync_copy(r1a, o1_hbm.at[pl.ds(off, _SC_CH)])

    return gather_kernel


_sc_gather_pair = _make_sc_gather2(3 * _HD, 2 * _HD)


def _edge_a_kernel(rowg, colg, ea, w1e, b1, h1, s, hsum, hsq):
    i = pl.program_id(0)
    rg = rowg[...]
    cg = colg[...]
    h = (
        rg[:, 0:_HD]
        + cg[:, 0:_HD]
        + jnp.dot(ea[...], w1e[...], preferred_element_type=jnp.float32)
        + b1[...]
    )
    h1[...] = h
    s[...] = jnp.sum(
        rg[:, _HD : 2 * _HD] * cg[:, _HD : 2 * _HD], axis=1, keepdims=True
    )

    @pl.when(i == 0)
    def _init():
        hsum[...] = jnp.zeros_like(hsum)
        hsq[...] = jnp.zeros_like(hsq)

    hsum[...] += jnp.sum(h, axis=0, keepdims=True)
    hsq[...] += jnp.sum(h * h, axis=0, keepdims=True)


def _edge_a(rowg, colg, ea, w1e, b1):
    return pl.pallas_call(
        _edge_a_kernel,
        grid=(_E // _BE,),
        in_specs=[
            pl.BlockSpec((_BE, 3 * _HD), lambda i: (i, 0)),
            pl.BlockSpec((_BE, 2 * _HD), lambda i: (i, 0)),
            pl.BlockSpec((_BE, _ED), lambda i: (i, 0)),
            pl.BlockSpec((_ED, _HD), lambda i: (0, 0)),
            pl.BlockSpec((1, _HD), lambda i: (0, 0)),
        ],
        out_specs=[
            pl.BlockSpec((_BE, _HD), lambda i: (i, 0)),
            pl.BlockSpec((_BE, 1), lambda i: (i, 0)),
            pl.BlockSpec((1, _HD), lambda i: (0, 0)),
            pl.BlockSpec((1, _HD), lambda i: (0, 0)),
        ],
        out_shape=[
            jax.ShapeDtypeStruct((_E, _HD), jnp.float32),
            jax.ShapeDtypeStruct((_E, 1), jnp.float32),
            jax.ShapeDtypeStruct((1, _HD), jnp.float32),
            jax.ShapeDtypeStruct((1, _HD), jnp.float32),
        ],
    )(rowg, colg, ea, w1e, b1)


def _edge_b_kernel(
    h1, hsum, hsq, w2e, b2e, wn1e, bn1, rowg, s, smax_g,
    eout, g, ex, gsum, gsq,
):
    i = pl.program_id(0)
    m = hsum[...] / _E
    v = hsq[...] / _E - m * m
    inv = jax.lax.rsqrt(v + _EPS)
    e1 = jnp.maximum((h1[...] - m) * inv, 0.0)
    eo = jnp.dot(e1, w2e[...], preferred_element_type=jnp.float32) + b2e[...]
    eout[...] = eo
    gb = (
        rowg[...][:, 2 * _HD :]
        + jnp.dot(eo, wn1e[...], preferred_element_type=jnp.float32)
        + bn1[...]
    )
    g[...] = gb
    ex[...] = jnp.exp(s[...] - smax_g[...])

    @pl.when(i == 0)
    def _init():
        gsum[...] = jnp.zeros_like(gsum)
        gsq[...] = jnp.zeros_like(gsq)

    gsum[...] += jnp.sum(gb, axis=0, keepdims=True)
    gsq[...] += jnp.sum(gb * gb, axis=0, keepdims=True)


def _edge_b(h1, hsum, hsq, w2e, b2e, wn1e, bn1, rowg, s, smax_g):
    return pl.pallas_call(
        _edge_b_kernel,
        grid=(_E // _BE,),
        in_specs=[
            pl.BlockSpec((_BE, _HD), lambda i: (i, 0)),
            pl.BlockSpec((1, _HD), lambda i: (0, 0)),
            pl.BlockSpec((1, _HD), lambda i: (0, 0)),
            pl.BlockSpec((_HD, _ED), lambda i: (0, 0)),
            pl.BlockSpec((1, _ED), lambda i: (0, 0)),
            pl.BlockSpec((_ED, _HD), lambda i: (0, 0)),
            pl.BlockSpec((1, _HD), lambda i: (0, 0)),
            pl.BlockSpec((_BE, 3 * _HD), lambda i: (i, 0)),
            pl.BlockSpec((_BE, 1), lambda i: (i, 0)),
            pl.BlockSpec((_BE, 1), lambda i: (i, 0)),
        ],
        out_specs=[
            pl.BlockSpec((_BE, _ED), lambda i: (i, 0)),
            pl.BlockSpec((_BE, _HD), lambda i: (i, 0)),
            pl.BlockSpec((_BE, 1), lambda i: (i, 0)),
            pl.BlockSpec((1, _HD), lambda i: (0, 0)),
            pl.BlockSpec((1, _HD), lambda i: (0, 0)),
        ],
        out_shape=[
            jax.ShapeDtypeStruct((_E, _ED), jnp.float32),
            jax.ShapeDtypeStruct((_E, _HD), jnp.float32),
            jax.ShapeDtypeStruct((_E, 1), jnp.float32),
            jax.ShapeDtypeStruct((1, _HD), jnp.float32),
            jax.ShapeDtypeStruct((1, _HD), jnp.float32),
        ],
    )(h1, hsum, hsq, w2e, b2e, wn1e, bn1, rowg, s, smax_g)


def _edge_c_kernel(g, gsum, gsq, wn2, bn2, ex, den_g, hw, attn):
    m = gsum[...] / _E
    v = gsq[...] / _E - m * m
    inv = jax.lax.rsqrt(v + _EPS)
    gn = jnp.maximum((g[...] - m) * inv, 0.0)
    h = jnp.dot(gn, wn2[...], preferred_element_type=jnp.float32) + bn2[...]
    a = ex[...] / den_g[...]
    attn[...] = a
    hw[...] = h * a


def _edge_c(g, gsum, gsq, wn2, bn2, ex, den_g):
    return pl.pallas_call(
        _edge_c_kernel,
        grid=(_E // _BE,),
        in_specs=[
            pl.BlockSpec((_BE, _HD), lambda i: (i, 0)),
            pl.BlockSpec((1, _HD), lambda i: (0, 0)),
            pl.BlockSpec((1, _HD), lambda i: (0, 0)),
            pl.BlockSpec((_HD, _HD), lambda i: (0, 0)),
            pl.BlockSpec((1, _HD), lambda i: (0, 0)),
            pl.BlockSpec((_BE, 1), lambda i: (i, 0)),
            pl.BlockSpec((_BE, 1), lambda i: (i, 0)),
        ],
        out_specs=[
            pl.BlockSpec((_BE, _HD), lambda i: (i, 0)),
            pl.BlockSpec((_BE, 1), lambda i: (i, 0)),
        ],
        out_shape=[
            jax.ShapeDtypeStruct((_E, _HD), jnp.float32),
            jax.ShapeDtypeStruct((_E, 1), jnp.float32),
        ],
    )(g, gsum, gsq, wn2, bn2, ex, den_g)


def _node_a_kernel(x, agg, wm1x, wm1a, bm1, t, tsum, tsq):
    i = pl.program_id(0)
    tb = (
        jnp.dot(x[...], wm1x[...], preferred_element_type=jnp.float32)
        + jnp.dot(agg[...], wm1a[...], preferred_element_type=jnp.float32)
        + bm1[...]
    )
    t[...] = tb

    @pl.when(i == 0)
    def _init():
        tsum[...] = jnp.zeros_like(tsum)
        tsq[...] = jnp.zeros_like(tsq)

    tsum[...] += jnp.sum(tb, axis=0, keepdims=True)
    tsq[...] += jnp.sum(tb * tb, axis=0, keepdims=True)


def _node_a(x, agg, wm1x, wm1a, bm1):
    return pl.pallas_call(
        _node_a_kernel,
        grid=(_N // _BN,),
        in_specs=[
            pl.BlockSpec((_BN, _ND), lambda i: (i, 0)),
            pl.BlockSpec((_BN, _HD), lambda i: (i, 0)),
            pl.BlockSpec((_ND, _HD), lambda i: (0, 0)),
            pl.BlockSpec((_HD, _HD), lambda i: (0, 0)),
            pl.BlockSpec((1, _HD), lambda i: (0, 0)),
        ],
        out_specs=[
            pl.BlockSpec((_BN, _HD), lambda i: (i, 0)),
            pl.BlockSpec((1, _HD), lambda i: (0, 0)),
            pl.BlockSpec((1, _HD), lambda i: (0, 0)),
        ],
        out_shape=[
            jax.ShapeDtypeStruct((_N, _HD), jnp.float32),
            jax.ShapeDtypeStruct((1, _HD), jnp.float32),
            jax.ShapeDtypeStruct((1, _HD), jnp.float32),
        ],
    )(x, agg, wm1x, wm1a, bm1)


def _node_b_kernel(t, tsum, tsq, wm2, bm2, xout):
    m = tsum[...] / _N
    v = tsq[...] / _N - m * m
    inv = jax.lax.rsqrt(v + _EPS)
    tn = jnp.maximum((t[...] - m) * inv, 0.0)
    xout[...] = (
        jnp.dot(tn, wm2[...], preferred_element_type=jnp.float32) + bm2[...]
    )


def _node_b(t, tsum, tsq, wm2, bm2):
    return pl.pallas_call(
        _node_b_kernel,
        grid=(_N // _BN,),
        in_specs=[
            pl.BlockSpec((_BN, _HD), lambda i: (i, 0)),
            pl.BlockSpec((1, _HD), lambda i: (0, 0)),
            pl.BlockSpec((1, _HD), lambda i: (0, 0)),
            pl.BlockSpec((_HD, _ND), lambda i: (0, 0)),
            pl.BlockSpec((1, _ND), lambda i: (0, 0)),
        ],
        out_specs=pl.BlockSpec((_BN, _ND), lambda i: (i, 0)),
        out_shape=jax.ShapeDtypeStruct((_N, _ND), jnp.float32),
    )(t, tsum, tsq, wm2, bm2)


def _proj_kernel(x, wp, bp, emb, xsum, gemb):
    i = pl.program_id(0)
    emb[...] = jnp.maximum(
        jnp.dot(x[...], wp[...], preferred_element_type=jnp.float32) + bp[...],
        0.0,
    )

    @pl.when(i == 0)
    def _init():
        xsum[...] = jnp.zeros_like(xsum)
        gemb[...] = jnp.zeros_like(gemb)

    xsum[...] += jnp.sum(x[...], axis=0, keepdims=True)

    @pl.when(i == pl.num_programs(0) - 1)
    def _final():
        gemb[...] = jnp.maximum(
            jnp.dot(
                xsum[...] / _N, wp[...], preferred_element_type=jnp.float32
            )
            + bp[...],
            0.0,
        )


def _proj(x, wp, bp):
    return pl.pallas_call(
        _proj_kernel,
        grid=(_N // _BN,),
        in_specs=[
            pl.BlockSpec((_BN, _ND), lambda i: (i, 0)),
            pl.BlockSpec((_ND, _OD), lambda i: (0, 0)),
            pl.BlockSpec((1, _OD), lambda i: (0, 0)),
        ],
        out_specs=[
            pl.BlockSpec((_BN, _OD), lambda i: (i, 0)),
            pl.BlockSpec((1, _ND), lambda i: (0, 0)),
            pl.BlockSpec((1, _OD), lambda i: (0, 0)),
        ],
        out_shape=[
            jax.ShapeDtypeStruct((_N, _OD), jnp.float32),
            jax.ShapeDtypeStruct((1, _ND), jnp.float32),
            jax.ShapeDtypeStruct((1, _OD), jnp.float32),
        ],
    )(x, wp, bp)


def _layer(x, edge_attr, lp, row, col):
    (w_e1, b_e1), (w_e2, b_e2) = lp["edge_mlp"]
    wq, bq = lp["attn_q"]
    wk, bk = lp["attn_k"]
    (w_n1, b_n1), (w_n2, b_n2) = lp["node_mlp1"]
    (w_m1, b_m1), (w_m2, b_m2) = lp["node_mlp2"]

    w1s = w_e1[:_ND]
    w1d = w_e1[_ND : 2 * _ND]
    w1e = w_e1[2 * _ND :]
    wn1x = w_n1[:_ND]
    wn1e = w_n1[_ND:]
    wm1x = w_m1[:_ND]
    wm1a = w_m1[_ND:]

    z = jnp.zeros((_HD,), jnp.float32)
    w_row = jnp.concatenate([w1s, wq, wn1x], axis=1)
    b_row = jnp.concatenate([z, bq, z])[None]
    w_col = jnp.concatenate([w1d, wk], axis=1)
    b_col = jnp.concatenate([z, bk])[None]
    xt_row = _node_transform(x, w_row, b_row)
    xt_col = _node_transform(x, w_col, b_col)

    rowg, colg = _sc_gather_pair(xt_row, xt_col, row, col)

    h1, s, hsum, hsq = _edge_a(rowg, colg, edge_attr, w1e, b_e1[None])

    s1 = s[:, 0]
    smax = jax.ops.segment_max(s1, row, num_segments=_N)
    smax_g = jnp.take(smax, row, axis=0)[:, None]

    eout, g, ex, gsum, gsq = _edge_b(
        h1, hsum, hsq, w_e2, b_e2[None], wn1e, b_n1[None], rowg, s, smax_g
    )

    den = jax.ops.segment_sum(ex[:, 0], row, num_segments=_N)
    den_g = jnp.take(den, row, axis=0)[:, None]

    hw, attn = _edge_c(g, gsum, gsq, w_n2, b_n2[None], ex, den_g)

    agg = jax.ops.segment_sum(hw, col, num_segments=_N)

    t, tsum, tsq = _node_a(x, agg, wm1x, wm1a, b_m1[None])
    x_out = _node_b(t, tsum, tsq, wm2=w_m2, bm2=b_m2[None])

    return x_out, eout, attn[:, 0]


def kernel(x, edge_index, edge_attr, params):
    row = edge_index[0].astype(jnp.int32)
    col = edge_index[1].astype(jnp.int32)
    attentions = []
    for lp in params["layers"]:
        x, edge_attr, attn = _layer(x, edge_attr, lp, row, col)
        attentions.append(attn)
    wp, bp = params["proj"]
    emb, _, gemb = _proj(x, wp, bp[None])
    u = params["layers"][-1]["global_mlp"][1][1][None, :]
    return (gemb, (emb, x, edge_attr, u, tuple(attentions)))
